# Initial kernel scaffold; baseline (speedup 1.0000x reference)
import jax
import jax.numpy as jnp
from jax.experimental import pallas as pl


def kernel(rays_o, rays_d, weights):
    raise NotImplementedError("write your pallas kernel here")


# speedup vs baseline: 36.5846x; 36.5846x over previous
import functools

import jax
import jax.numpy as jnp
from jax import lax
from jax.experimental import pallas as pl
from jax.experimental.pallas import tpu as pltpu
from jax.experimental.pallas import tpu_sc as plsc

N_RAYS = 65536
N_BINS = 256
N_SAMPLES = 256
L = 16
NC, NSUB = 2, 16
NW = NC * NSUB
RAYS_PER_W = N_RAYS // NW
RB = 128
NBLK = RAYS_PER_W // RB
NCH = N_BINS // L
F32 = jnp.float32
I32 = jnp.int32


def _body(ox_h, oy_h, oz_h, dx_h, dy_h, dz_h, w_h, out_h,
          ox_v, oy_v, oz_v, dx_v, dy_v, dz_v,
          near_v, hs_v, w_v, out_v, c_v, hist_v):
    wid = lax.axis_index("c") * NSUB + lax.axis_index("s")
    ones_i = jnp.full((L,), 1, I32)
    zeros_i = jnp.full((L,), 0, I32)
    iota_f = lax.iota(I32, L).astype(F32)

    def block(blk, _):
        rbase = wid * RAYS_PER_W + blk * RB
        pltpu.sync_copy(w_h.at[pl.ds(rbase, RB)], w_v)
        pltpu.sync_copy(ox_h.at[pl.ds(rbase, RB)], ox_v)
        pltpu.sync_copy(oy_h.at[pl.ds(rbase, RB)], oy_v)
        pltpu.sync_copy(oz_h.at[pl.ds(rbase, RB)], oz_v)
        pltpu.sync_copy(dx_h.at[pl.ds(rbase, RB)], dx_v)
        pltpu.sync_copy(dy_h.at[pl.ds(rbase, RB)], dy_v)
        pltpu.sync_copy(dz_h.at[pl.ds(rbase, RB)], dz_v)

        for g in range(RB // L):
            sl = pl.ds(g * L, L)
            lo = None
            hi = None
            for o_ref, d_ref in ((ox_v, dx_v), (oy_v, dy_v), (oz_v, dz_v)):
                o = o_ref[sl]
                d = d_ref[sl] + F32(1e-15)
                tmin = (F32(-2.0) - o) / d
                tmax = (F32(2.0) - o) / d
                a_lo = jnp.where(tmin < tmax, tmin, tmax)
                a_hi = jnp.where(tmin > tmax, tmin, tmax)
                lo = a_lo if lo is None else jnp.maximum(lo, a_lo)
                hi = a_hi if hi is None else jnp.minimum(hi, a_hi)
            bad = hi < lo
            nr = jnp.where(bad, F32(1e9), lo)
            fr = jnp.where(bad, F32(1e9), hi)
            nr = jnp.maximum(nr, F32(0.05))
            near_v[sl] = nr
            hs_v[sl] = (fr - nr) * F32(1.0 / 256.0)

        def ray(r, _):
            for cc in range(17):
                hist_v[pl.ds(cc * L, L)] = zeros_i

            totals = []
            for cc in range(NCH):
                v = w_v[r, pl.ds(cc * L, L)] + F32(1e-5)
                totals.append(jnp.sum(v))
            off = F32(0.0)
            offs = []
            for cc in range(NCH):
                offs.append(off)
                off = off + totals[cc]
            s_tot = off

            for cc in range(NCH):
                v = w_v[r, pl.ds(cc * L, L)] + F32(1e-5)
                cs = (plsc.cumsum(v) + offs[cc]) / s_tot
                c_v[pl.ds(cc * L, L)] = cs
                m = cs * F32(256.0) - F32(0.5)
                ti = m.astype(I32)
                cl = ti + jnp.where(m > ti.astype(F32), 1, 0)
                sidx = jnp.minimum(jnp.maximum(cl, 0), 256)
                plsc.addupdate_scatter(hist_v, [sidx], ones_i)

            ridx = jnp.full((L,), r, I32)
            near_s = plsc.load_gather(near_v, [ridx])
            hs_s = plsc.load_gather(hs_v, [ridx])

            hoff = I32(0)
            for cc in range(NCH):
                h = hist_v[pl.ds(cc * L, L)]
                b = plsc.cumsum(h) + hoff
                hoff = hoff + jnp.sum(h)
                ib = jnp.maximum(b - 1, 0)
                cgb = plsc.load_gather(c_v, [ib])
                cgb = jnp.where(b == 0, F32(0.0), cgb)
                ia = jnp.minimum(b, 255)
                cga = plsc.load_gather(c_v, [ia])
                denom = cga - cgb
                denom = jnp.where(denom < F32(1e-5), F32(1.0), denom)
                u = (iota_f + F32(cc * L) + F32(0.5)) * F32(1.0 / 256.0)
                t = (u - cgb) / denom
                bf = b.astype(F32)
                af = jnp.minimum(b + 1, 256).astype(F32)
                y = bf + t * (af - bf)
                out_v[r, pl.ds(cc * L, L)] = near_s + hs_s * y
            return 0

        lax.fori_loop(0, RB, ray, 0)
        pltpu.sync_copy(out_v, out_h.at[pl.ds(rbase, RB)])
        return 0

    lax.fori_loop(0, NBLK, block, 0)


@jax.jit
def kernel(rays_o, rays_d, weights):
    mesh = plsc.VectorSubcoreMesh(core_axis_name="c", subcore_axis_name="s")
    k = functools.partial(
        pl.kernel,
        out_type=jax.ShapeDtypeStruct((N_RAYS, N_SAMPLES), F32),
        mesh=mesh,
        compiler_params=pltpu.CompilerParams(needs_layout_passes=False),
        scratch_types=[
            pltpu.VMEM((RB,), F32),
            pltpu.VMEM((RB,), F32),
            pltpu.VMEM((RB,), F32),
            pltpu.VMEM((RB,), F32),
            pltpu.VMEM((RB,), F32),
            pltpu.VMEM((RB,), F32),
            pltpu.VMEM((RB,), F32),
            pltpu.VMEM((RB,), F32),
            pltpu.VMEM((RB, N_BINS), F32),
            pltpu.VMEM((RB, N_SAMPLES), F32),
            pltpu.VMEM((N_BINS,), F32),
            pltpu.VMEM((272,), I32),
        ],
    )(_body)
    return k(
        rays_o[:, 0].astype(F32), rays_o[:, 1].astype(F32), rays_o[:, 2].astype(F32),
        rays_d[:, 0].astype(F32), rays_d[:, 1].astype(F32), rays_d[:, 2].astype(F32),
        weights.astype(F32),
    )


# speedup vs baseline: 37.2031x; 1.0169x over previous
import functools

import jax
import jax.numpy as jnp
from jax import lax
from jax.experimental import pallas as pl
from jax.experimental.pallas import tpu as pltpu
from jax.experimental.pallas import tpu_sc as plsc

N_RAYS = 65536
N_BINS = 256
N_SAMPLES = 256
L = 16
NC, NSUB = 2, 16
NW = NC * NSUB
RAYS_PER_W = N_RAYS // NW
RB = 128
NBLK = RAYS_PER_W // RB
NCH = N_BINS // L
F32 = jnp.float32
I32 = jnp.int32


def _body(ox_h, oy_h, oz_h, dx_h, dy_h, dz_h, w_h, out_h,
          ox_v, oy_v, oz_v, dx_v, dy_v, dz_v,
          near_v, hs_v, w_v, out_v, c0_v, c1_v, h0_v, h1_v):
    wid = lax.axis_index("c") * NSUB + lax.axis_index("s")
    ones_i = jnp.full((L,), 1, I32)
    zeros_i = jnp.full((L,), 0, I32)
    iota_f = lax.iota(I32, L).astype(F32)

    def block(blk, _):
        rbase = wid * RAYS_PER_W + blk * RB
        pltpu.sync_copy(w_h.at[pl.ds(rbase, RB)], w_v)
        pltpu.sync_copy(ox_h.at[pl.ds(rbase, RB)], ox_v)
        pltpu.sync_copy(oy_h.at[pl.ds(rbase, RB)], oy_v)
        pltpu.sync_copy(oz_h.at[pl.ds(rbase, RB)], oz_v)
        pltpu.sync_copy(dx_h.at[pl.ds(rbase, RB)], dx_v)
        pltpu.sync_copy(dy_h.at[pl.ds(rbase, RB)], dy_v)
        pltpu.sync_copy(dz_h.at[pl.ds(rbase, RB)], dz_v)

        for g in range(RB // L):
            sl = pl.ds(g * L, L)
            lo = None
            hi = None
            for o_ref, d_ref in ((ox_v, dx_v), (oy_v, dy_v), (oz_v, dz_v)):
                o = o_ref[sl]
                d = d_ref[sl] + F32(1e-15)
                tmin = (F32(-2.0) - o) / d
                tmax = (F32(2.0) - o) / d
                a_lo = jnp.where(tmin < tmax, tmin, tmax)
                a_hi = jnp.where(tmin > tmax, tmin, tmax)
                lo = a_lo if lo is None else jnp.maximum(lo, a_lo)
                hi = a_hi if hi is None else jnp.minimum(hi, a_hi)
            bad = hi < lo
            nr = jnp.where(bad, F32(1e9), lo)
            fr = jnp.where(bad, F32(1e9), hi)
            nr = jnp.maximum(nr, F32(0.05))
            near_v[sl] = nr
            hs_v[sl] = (fr - nr) * F32(1.0 / 256.0)

        iota_i = lax.iota(I32, L)

        def ray(i, _):
            rr = (i * 2, i * 2 + 1)
            c_v = (c0_v, c1_v)
            hist_v = (h0_v, h1_v)
            for p in range(2):
                for cc in range(17):
                    hist_v[p][pl.ds(cc * L, L)] = zeros_i

            offs = [[], []]
            s_tot = [None, None]
            for p in range(2):
                totals = [
                    jnp.sum(w_v[rr[p], pl.ds(cc * L, L)] + F32(1e-5))
                    for cc in range(NCH)
                ]
                off = F32(0.0)
                for cc in range(NCH):
                    offs[p].append(off)
                    off = off + totals[cc]
                s_tot[p] = off

            for p in range(2):
                c_v[p][pl.ds(0, L)] = jnp.full((L,), 0.0, F32)
                for cc in range(NCH):
                    v = w_v[rr[p], pl.ds(cc * L, L)] + F32(1e-5)
                    cs = (plsc.cumsum(v) + offs[p][cc]) / s_tot[p]
                    plsc.store_scatter(c_v[p], [iota_i + (cc * L + 1)], cs)
                    m = cs * F32(256.0) - F32(0.5)
                    ti = m.astype(I32)
                    cl = ti + jnp.where(m > ti.astype(F32), 1, 0)
                    sidx = jnp.minimum(jnp.maximum(cl, 0), 256)
                    plsc.addupdate_scatter(hist_v[p], [sidx], ones_i)

            for p in range(2):
                ridx = jnp.full((L,), rr[p], I32)
                near_s = plsc.load_gather(near_v, [ridx])
                hs_s = plsc.load_gather(hs_v, [ridx])
                hoff = I32(0)
                for cc in range(NCH):
                    h = hist_v[p][pl.ds(cc * L, L)]
                    b = plsc.cumsum(h) + hoff
                    hoff = hoff + jnp.sum(h)
                    cgb = plsc.load_gather(c_v[p], [b])
                    ia = jnp.minimum(b + 1, 256)
                    cga = plsc.load_gather(c_v[p], [ia])
                    denom = cga - cgb
                    denom = jnp.where(denom < F32(1e-5), F32(1.0), denom)
                    u = (iota_f + F32(cc * L) + F32(0.5)) * F32(1.0 / 256.0)
                    t = (u - cgb) / denom
                    y = b.astype(F32) + t * (ia - b).astype(F32)
                    out_v[rr[p], pl.ds(cc * L, L)] = near_s + hs_s * y
            return 0

        lax.fori_loop(0, RB // 2, ray, 0)
        pltpu.sync_copy(out_v, out_h.at[pl.ds(rbase, RB)])
        return 0

    lax.fori_loop(0, NBLK, block, 0)


@jax.jit
def kernel(rays_o, rays_d, weights):
    mesh = plsc.VectorSubcoreMesh(core_axis_name="c", subcore_axis_name="s")
    k = functools.partial(
        pl.kernel,
        out_type=jax.ShapeDtypeStruct((N_RAYS, N_SAMPLES), F32),
        mesh=mesh,
        compiler_params=pltpu.CompilerParams(needs_layout_passes=False),
        scratch_types=[
            pltpu.VMEM((RB,), F32),
            pltpu.VMEM((RB,), F32),
            pltpu.VMEM((RB,), F32),
            pltpu.VMEM((RB,), F32),
            pltpu.VMEM((RB,), F32),
            pltpu.VMEM((RB,), F32),
            pltpu.VMEM((RB,), F32),
            pltpu.VMEM((RB,), F32),
            pltpu.VMEM((RB, N_BINS), F32),
            pltpu.VMEM((RB, N_SAMPLES), F32),
            pltpu.VMEM((272,), F32),
            pltpu.VMEM((272,), F32),
            pltpu.VMEM((272,), I32),
            pltpu.VMEM((272,), I32),
        ],
    )(_body)
    return k(
        rays_o[:, 0].astype(F32), rays_o[:, 1].astype(F32), rays_o[:, 2].astype(F32),
        rays_d[:, 0].astype(F32), rays_d[:, 1].astype(F32), rays_d[:, 2].astype(F32),
        weights.astype(F32),
    )
